# baseline (speedup 1.0000x reference)
import functools

import jax
import jax.numpy as jnp
from jax import lax
from jax.experimental import pallas as pl
from jax.experimental.pallas import tpu as pltpu
from jax.experimental.pallas import tpu_sc as plsc

D = 256
B = 1024
P = 11
L = 16
DCH = D // L
NC, NS = 2, 16
NW = NC * NS
BPW = B // NW
CB = 8
NCH = BPW // CB
RP = CB * P
SW = 2 * P * L

_mesh = plsc.VectorSubcoreMesh(
    core_axis_name="c", subcore_axis_name="s", num_cores=NC, num_subcores=NS
)


@functools.partial(
    pl.kernel,
    out_type=jax.ShapeDtypeStruct((B, SW), jnp.float32),
    mesh=_mesh,
    scratch_types=[
        pltpu.VMEM((NCH, CB), jnp.int32),
        pltpu.VMEM((NCH, RP), jnp.int32),
        pltpu.VMEM((NCH, RP), jnp.int32),
        pltpu.VMEM((2, CB, D), jnp.float32),
        pltpu.VMEM((2, RP, D), jnp.float32),
        pltpu.VMEM((2, RP, D), jnp.float32),
        pltpu.VMEM((BPW, SW), jnp.float32),
        pltpu.SemaphoreType.DMA,
        pltpu.SemaphoreType.DMA,
    ],
    compiler_params=pltpu.CompilerParams(needs_layout_passes=False),
)
def _sc_scores(emb, xid, pid, nid, out, xv, pv, nv, urows, prows, nrows, sv,
               sem0, sem1):
    wid = lax.axis_index("s") * NC + lax.axis_index("c")
    pltpu.sync_copy(xid.at[wid], xv)
    pltpu.sync_copy(pid.at[wid], pv)
    pltpu.sync_copy(nid.at[wid], nv)
    sems = (sem0, sem1)

    def start(c):
        t = c % 2
        return (
            pltpu.async_copy(emb.at[xv.at[c]], urows.at[t], sems[t]),
            pltpu.async_copy(emb.at[pv.at[c]], prows.at[t], sems[t]),
            pltpu.async_copy(emb.at[nv.at[c]], nrows.at[t], sems[t]),
        )

    pend = start(0)
    for c in range(NCH):
        nxt = start(c + 1) if c + 1 < NCH else None
        for dsc in pend:
            dsc.wait()
        t = c % 2

        def b_body(bl, _, c=c, t=t):
            base = bl * P
            acc = [jnp.zeros((L,), jnp.float32) for _ in range(2 * P)]
            for k in range(DCH):
                uk = urows[t, bl, pl.ds(k * L, L)]
                for j in range(P):
                    acc[j] = acc[j] + uk * prows[t, base + j, pl.ds(k * L, L)]
                    acc[P + j] = (
                        acc[P + j] + uk * nrows[t, base + j, pl.ds(k * L, L)]
                    )
            gb = c * CB + bl
            for j in range(2 * P):
                sv[gb, pl.ds(j * L, L)] = acc[j]
            return 0

        lax.fori_loop(0, CB, b_body, 0)
        pend = nxt

    pltpu.sync_copy(sv, out.at[pl.ds(wid * BPW, BPW)])


def _tc_loss(slab):
    def body(s_ref, o_ref):
        s = s_ref[...]
        ci = lax.broadcasted_iota(jnp.int32, (SW, 2 * P), 0) // L
        cj = lax.broadcasted_iota(jnp.int32, (SW, 2 * P), 1)
        fold = (ci == cj).astype(jnp.float32)
        scores = jnp.dot(s, fold, precision=lax.Precision.HIGHEST,
                         preferred_element_type=jnp.float32)
        lt = jnp.mean(jax.nn.log_sigmoid(scores[:, 0:P]), axis=1)
        sl = jnp.mean(jax.nn.log_sigmoid(-scores[:, P:2 * P]), axis=1)
        o_ref[...] = -(lt + sl)

    return pl.pallas_call(
        body, out_shape=jax.ShapeDtypeStruct((B,), jnp.float32)
    )(slab)


def kernel(homo_emb, x_id, pos_id, neg_id, batch_num=0):
    xid = jnp.asarray(x_id, jnp.int32).reshape(NW, NCH, CB)
    pid = jnp.asarray(pos_id, jnp.int32).reshape(NW, NCH, RP)
    nid = jnp.asarray(neg_id, jnp.int32).reshape(NW, NCH, RP)
    slab = _sc_scores(homo_emb, xid, pid, nid)
    return _tc_loss(slab)


# speedup vs baseline: 1.1483x; 1.1483x over previous
import functools

import jax
import jax.numpy as jnp
from jax import lax
from jax.experimental import pallas as pl
from jax.experimental.pallas import tpu as pltpu
from jax.experimental.pallas import tpu_sc as plsc

D = 256
B = 1024
P = 11
L = 16
DCH = D // L
NC, NS = 2, 16
NW = NC * NS
BPW = B // NW
CB = 8
NCH = BPW // CB
RP = CB * P
SW = 2 * P * L

_mesh = plsc.VectorSubcoreMesh(
    core_axis_name="c", subcore_axis_name="s", num_cores=NC, num_subcores=NS
)


@functools.partial(
    pl.kernel,
    out_type=jax.ShapeDtypeStruct((B, SW), jnp.float32),
    mesh=_mesh,
    scratch_types=[
        pltpu.VMEM((NCH, CB), jnp.int32),
        pltpu.VMEM((NCH, RP), jnp.int32),
        pltpu.VMEM((NCH, RP), jnp.int32),
        pltpu.VMEM((2, CB, D), jnp.float32),
        pltpu.VMEM((2, RP, D), jnp.float32),
        pltpu.VMEM((2, RP, D), jnp.float32),
        pltpu.VMEM((BPW, SW), jnp.float32),
        pltpu.SemaphoreType.DMA,
        pltpu.SemaphoreType.DMA,
    ],
    compiler_params=pltpu.CompilerParams(needs_layout_passes=False),
)
def _sc_scores(emb, xid, pid, nid, out, xv, pv, nv, urows, prows, nrows, sv,
               sem0, sem1):
    wid = lax.axis_index("s") * NC + lax.axis_index("c")
    pltpu.sync_copy(xid.at[wid], xv)
    pltpu.sync_copy(pid.at[wid], pv)
    pltpu.sync_copy(nid.at[wid], nv)
    sems = (sem0, sem1)

    def start(c):
        t = c % 2
        return (
            pltpu.async_copy(emb.at[xv.at[c]], urows.at[t], sems[t]),
            pltpu.async_copy(emb.at[pv.at[c]], prows.at[t], sems[t]),
            pltpu.async_copy(emb.at[nv.at[c]], nrows.at[t], sems[t]),
        )

    pend = start(0)
    for c in range(NCH):
        nxt = start(c + 1) if c + 1 < NCH else None
        for dsc in pend:
            dsc.wait()
        t = c % 2

        def b_body(bl, _, c=c, t=t):
            base = bl * P
            gb = c * CB + bl
            for side, rows in ((0, prows), (1, nrows)):
                acc = [jnp.zeros((L,), jnp.float32) for _ in range(P)]
                for k in range(DCH):
                    uk = urows[t, bl, pl.ds(k * L, L)]
                    for j in range(P):
                        acc[j] = acc[j] + uk * rows[t, base + j, pl.ds(k * L, L)]
                for j in range(P):
                    sv[gb, pl.ds((side * P + j) * L, L)] = acc[j]
            return 0

        lax.fori_loop(0, CB, b_body, 0)
        pend = nxt

    pltpu.sync_copy(sv, out.at[pl.ds(wid * BPW, BPW)])


def _tc_loss(slab):
    def body(s_ref, o_ref):
        s = s_ref[...]
        ci = lax.broadcasted_iota(jnp.int32, (SW, 2 * P), 0) // L
        cj = lax.broadcasted_iota(jnp.int32, (SW, 2 * P), 1)
        fold = (ci == cj).astype(jnp.float32)
        scores = jnp.dot(s, fold, precision=lax.Precision.HIGHEST,
                         preferred_element_type=jnp.float32)
        lt = jnp.mean(jax.nn.log_sigmoid(scores[:, 0:P]), axis=1)
        sl = jnp.mean(jax.nn.log_sigmoid(-scores[:, P:2 * P]), axis=1)
        o_ref[...] = -(lt + sl)

    return pl.pallas_call(
        body, out_shape=jax.ShapeDtypeStruct((B,), jnp.float32)
    )(slab)


def kernel(homo_emb, x_id, pos_id, neg_id, batch_num=0):
    xid = jnp.asarray(x_id, jnp.int32).reshape(NW, NCH, CB)
    pid = jnp.asarray(pos_id, jnp.int32).reshape(NW, NCH, RP)
    nid = jnp.asarray(neg_id, jnp.int32).reshape(NW, NCH, RP)
    slab = _sc_scores(homo_emb, xid, pid, nid)
    return _tc_loss(slab)
